# Initial kernel scaffold; baseline (speedup 1.0000x reference)
import jax
import jax.numpy as jnp
from jax.experimental import pallas as pl


def kernel(x, edge_index, pos, W_lin1, b_lin1, W_upd1, W_lin2, b_lin2, W_upd2, W_lin3, b_lin3, W_upd3, bn_gamma, bn_beta, W_out, b_out):
    raise NotImplementedError("write your pallas kernel here")


# speedup vs baseline: 3.1988x; 3.1988x over previous
import functools

import jax
import jax.numpy as jnp
from jax import lax
from jax.experimental import pallas as pl
from jax.experimental.pallas import tpu as pltpu
import jax.experimental.pallas.tpu_sc as plsc

N = 10000
E = 320000
D = 128
DOUT = 40

NC = 2
NS = 16
NW = NC * NS
CH = 128
CHUNKS = 79
EPW = CH * CHUNKS
EPAD = EPW * NW
NPAD = 10240
RPT = NPAD // NS
CW = 16

_f32 = jnp.float32
_i32 = jnp.int32


def _zero_vmem_2d(buf, rows, width):
    zero16 = jnp.zeros((16,), _f32)

    def body(i, _):
        r = i // (width // 16)
        c = (i % (width // 16)) * 16
        buf[r, pl.ds(c, 16)] = zero16
        return 0

    lax.fori_loop(0, rows * (width // 16), body, 0)


def _sc_scatter_body(z_hbm, row_hbm, col_hbm, acc_out, ridx_v, cidx_v, gbuf,
                     acc_sh, gsem):
    cid = lax.axis_index("c")
    sid = lax.axis_index("s")
    wid = cid * NS + sid

    _zero_vmem_2d(gbuf, CH, D)
    for k in range(RPT // CH):
        pltpu.sync_copy(gbuf, acc_sh.at[pl.ds(sid * RPT + k * CH, CH)])
    plsc.subcore_barrier()

    base_w = wid * EPW

    def chunk(i, _):
        base = base_w + i * CH
        pltpu.sync_copy(row_hbm.at[pl.ds(base, CH)], ridx_v)
        pltpu.async_copy(z_hbm.at[ridx_v], gbuf, gsem).wait()
        pltpu.sync_copy(col_hbm.at[pl.ds(base, CH)], cidx_v)
        pltpu.sync_copy(gbuf, acc_sh.at[cidx_v], add=True)
        return 0

    lax.fori_loop(0, CHUNKS, chunk, 0)
    plsc.subcore_barrier()

    pltpu.sync_copy(acc_sh.at[pl.ds(sid * RPT, RPT)],
                    acc_out.at[cid, pl.ds(sid * RPT, RPT)])


def _sc_count_body(col_hbm, cnt_out, cidx_v, ones_v, zbuf, cnt_sh):
    cid = lax.axis_index("c")
    sid = lax.axis_index("s")
    wid = cid * NS + sid

    _zero_vmem_2d(zbuf, CH, D)
    one16 = jnp.ones((16,), _f32)

    def fill_ones(i, _):
        r = i // (D // 16)
        c = (i % (D // 16)) * 16
        ones_v[r, pl.ds(c, 16)] = one16
        return 0

    lax.fori_loop(0, CH * (D // 16), fill_ones, 0)

    for k in range(RPT // CH):
        pltpu.sync_copy(zbuf, cnt_sh.at[pl.ds(sid * RPT + k * CH, CH)])
    plsc.subcore_barrier()

    base_w = wid * EPW

    def chunk(i, _):
        base = base_w + i * CH
        pltpu.sync_copy(col_hbm.at[pl.ds(base, CH)], cidx_v)
        pltpu.sync_copy(ones_v, cnt_sh.at[cidx_v], add=True)
        return 0

    lax.fori_loop(0, CHUNKS, chunk, 0)
    plsc.subcore_barrier()

    pltpu.sync_copy(cnt_sh.at[pl.ds(sid * RPT, RPT)],
                    cnt_out.at[cid, pl.ds(sid * RPT, RPT)])


@functools.lru_cache(maxsize=None)
def _sc_kernels():
    mesh = plsc.VectorSubcoreMesh(
        core_axis_name="c", subcore_axis_name="s",
        num_cores=NC, num_subcores=NS)

    scatter = pl.kernel(
        _sc_scatter_body,
        out_type=jax.ShapeDtypeStruct((NC, NPAD, D), _f32),
        mesh=mesh,
        scratch_types=[
            pltpu.VMEM((CH,), _i32),
            pltpu.VMEM((CH,), _i32),
            pltpu.VMEM((CH, D), _f32),
            pltpu.VMEM_SHARED((NPAD, D), _f32),
            pltpu.SemaphoreType.DMA,
        ],
    )

    count = pl.kernel(
        _sc_count_body,
        out_type=jax.ShapeDtypeStruct((NC, NPAD, D), _f32),
        mesh=mesh,
        scratch_types=[
            pltpu.VMEM((CH,), _i32),
            pltpu.VMEM((CH, D), _f32),
            pltpu.VMEM((CH, D), _f32),
            pltpu.VMEM_SHARED((NPAD, D), _f32),
        ],
    )
    return scatter, count


def _tc_lin_body(x_ref, w_ref, b_ref, z_ref):
    z_ref[...] = jnp.maximum(
        jnp.dot(x_ref[...], w_ref[...], preferred_element_type=_f32)
        + b_ref[...], 0.0)


def _tc_update_body(acc_ref, cnt_ref, x_ref, wu_ref, g_ref, be_ref,
                    wn_ref, bn_ref, h_ref, z_ref):
    acc = acc_ref[0, :N, :] + acc_ref[1, :N, :]
    cnt = cnt_ref[0, :N, :1] + cnt_ref[1, :N, :1]
    aggr = acc / jnp.maximum(cnt, 1.0)
    u = jnp.maximum(
        jnp.dot(aggr, wu_ref[:D, :], preferred_element_type=_f32)
        + jnp.dot(x_ref[...], wu_ref[D:, :], preferred_element_type=_f32),
        0.0)
    mu = jnp.mean(u, axis=0, keepdims=True)
    var = jnp.mean((u - mu) * (u - mu), axis=0, keepdims=True)
    h = jnp.maximum(
        g_ref[...] * (u - mu) / jnp.sqrt(var + 1e-5) + be_ref[...], 0.0)
    h_ref[...] = h
    z_ref[...] = jnp.maximum(
        jnp.dot(h, wn_ref[...], preferred_element_type=_f32) + bn_ref[...],
        0.0)


def _tc_final_body(acc_ref, cnt_ref, x_ref, wu_ref, wo_ref, bo_ref, o_ref):
    acc = acc_ref[0, :N, :] + acc_ref[1, :N, :]
    cnt = cnt_ref[0, :N, :1] + cnt_ref[1, :N, :1]
    aggr = acc / jnp.maximum(cnt, 1.0)
    u = jnp.maximum(
        jnp.dot(aggr, wu_ref[:D, :], preferred_element_type=_f32)
        + jnp.dot(x_ref[...], wu_ref[D:, :], preferred_element_type=_f32),
        0.0)
    o_ref[...] = jnp.dot(u, wo_ref[...], preferred_element_type=_f32) \
        + bo_ref[...]


_tc_lin = pl.pallas_call(
    _tc_lin_body, out_shape=jax.ShapeDtypeStruct((N, D), _f32))

_tc_update = pl.pallas_call(
    _tc_update_body,
    out_shape=[jax.ShapeDtypeStruct((N, D), _f32),
               jax.ShapeDtypeStruct((N, D), _f32)])

_tc_final = pl.pallas_call(
    _tc_final_body, out_shape=jax.ShapeDtypeStruct((N, D), _f32))


def kernel(x, edge_index, pos,
           W_lin1, b_lin1, W_upd1,
           W_lin2, b_lin2, W_upd2,
           W_lin3, b_lin3, W_upd3,
           bn_gamma, bn_beta, W_out, b_out):
    row = edge_index[0].astype(_i32)
    col = edge_index[1].astype(_i32)
    npad = EPAD - E
    rowp = jnp.concatenate([row, jnp.zeros((npad,), _i32)])
    colp = jnp.concatenate([col, jnp.full((npad,), N, _i32)])

    b1 = b_lin1.reshape(1, D)
    b2 = b_lin2.reshape(1, D)
    b3 = b_lin3.reshape(1, D)
    g = bn_gamma.reshape(1, D)
    be = bn_beta.reshape(1, D)
    wo = jnp.zeros((D, D), _f32).at[:, :DOUT].set(W_out)
    bo = jnp.zeros((1, D), _f32).at[0, :DOUT].set(b_out)

    _sc_scatter, _sc_count = _sc_kernels()
    cnt = _sc_count(colp)

    z1 = _tc_lin(x, W_lin1, b1)
    acc1 = _sc_scatter(z1, rowp, colp)
    h1, z2 = _tc_update(acc1, cnt, x, W_upd1, g, be, W_lin2, b2)
    acc2 = _sc_scatter(z2, rowp, colp)
    h2, z3 = _tc_update(acc2, cnt, h1, W_upd2, g, be, W_lin3, b3)
    acc3 = _sc_scatter(z3, rowp, colp)
    out = _tc_final(acc3, cnt, h2, W_upd3, wo, bo)
    return out[:, :DOUT]
